# Initial kernel scaffold; baseline (speedup 1.0000x reference)
import functools

import jax
import jax.numpy as jnp
from jax import lax
from jax.experimental import pallas as pl
from jax.experimental.pallas import tpu as pltpu
from jax.experimental.pallas import tpu_sc as plsc

_N = 10000
_E = 320000
_H = 128
_G = 64

_NC = 2
_NS = 16
_NW = _NC * _NS
_EPW = _E // _NW
_CH = 80
_NCHUNK = _EPW // _CH
_RPT = _N // _NS
_DPT = 624


def _sc_agg_body(with_deg, x_hbm, src_hbm, dst_hbm, ones_hbm, z2d_hbm, z1d_hbm,
                 agg_hbm, deg_hbm, idx_s, idx_d, rows, ones_v,
                 shared_agg, shared_deg, sem):
  c = lax.axis_index("c")
  s = lax.axis_index("s")
  wid = c * _NS + s

  pltpu.sync_copy(z2d_hbm, shared_agg.at[pl.ds(s * _RPT, _RPT)])
  if with_deg:
    pltpu.sync_copy(z1d_hbm.at[pl.ds(0, _DPT)],
                    shared_deg.at[pl.ds(s * _DPT, _DPT)])

    @pl.when(s == 0)
    def _():
      pltpu.sync_copy(z1d_hbm.at[pl.ds(0, _N - _NS * _DPT)],
                      shared_deg.at[pl.ds(_NS * _DPT, _N - _NS * _DPT)])

    pltpu.sync_copy(ones_hbm, ones_v)
  plsc.subcore_barrier()

  base = wid * _EPW

  def chunk(i, carry):
    off = base + i * _CH
    pltpu.sync_copy(src_hbm.at[pl.ds(off, _CH)], idx_s)
    pltpu.sync_copy(dst_hbm.at[pl.ds(off, _CH)], idx_d)
    pltpu.async_copy(x_hbm.at[idx_s], rows, sem).wait()
    pltpu.sync_copy(rows, shared_agg.at[idx_d], add=True)
    if with_deg:
      pltpu.sync_copy(ones_v, shared_deg.at[idx_d], add=True)
    return carry

  lax.fori_loop(0, _NCHUNK, chunk, 0)
  plsc.subcore_barrier()

  pltpu.sync_copy(shared_agg.at[pl.ds(s * _RPT, _RPT)],
                  agg_hbm.at[c, pl.ds(s * _RPT, _RPT)])
  if with_deg:
    pltpu.sync_copy(shared_deg.at[pl.ds(s * _DPT, _DPT)],
                    deg_hbm.at[c, pl.ds(s * _DPT, _DPT)])

    @pl.when(s == 0)
    def _():
      pltpu.sync_copy(shared_deg.at[pl.ds(_NS * _DPT, _N - _NS * _DPT)],
                      deg_hbm.at[c, pl.ds(_NS * _DPT, _N - _NS * _DPT)])


def _sc_agg_deg_body(x_hbm, src_hbm, dst_hbm, ones_hbm, z2d_hbm, z1d_hbm,
                     agg_hbm, deg_hbm, idx_s, idx_d, rows, ones_v,
                     shared_agg, shared_deg, sem):
  _sc_agg_body(True, x_hbm, src_hbm, dst_hbm, ones_hbm, z2d_hbm, z1d_hbm,
               agg_hbm, deg_hbm, idx_s, idx_d, rows, ones_v,
               shared_agg, shared_deg, sem)


def _sc_agg_nodeg_body(x_hbm, src_hbm, dst_hbm, ones_hbm, z2d_hbm, z1d_hbm,
                       agg_hbm, idx_s, idx_d, rows, ones_v,
                       shared_agg, shared_deg, sem):
  _sc_agg_body(False, x_hbm, src_hbm, dst_hbm, ones_hbm, z2d_hbm, z1d_hbm,
               agg_hbm, None, idx_s, idx_d, rows, ones_v,
               shared_agg, shared_deg, sem)


def _sc_scratch():
  return [
      pltpu.VMEM((_CH,), jnp.int32),
      pltpu.VMEM((_CH,), jnp.int32),
      pltpu.VMEM((_CH, _H), jnp.float32),
      pltpu.VMEM((_CH,), jnp.float32),
      pltpu.VMEM_SHARED((_N, _H), jnp.float32),
      pltpu.VMEM_SHARED((_N,), jnp.float32),
      pltpu.SemaphoreType.DMA,
  ]


_sc_mesh = plsc.VectorSubcoreMesh(core_axis_name="c", subcore_axis_name="s",
                                  num_cores=_NC, num_subcores=_NS)

_sc_agg_deg = pl.kernel(
    _sc_agg_deg_body,
    out_type=[jax.ShapeDtypeStruct((_NC, _N, _H), jnp.float32),
              jax.ShapeDtypeStruct((_NC, _N), jnp.float32)],
    mesh=_sc_mesh,
    scratch_types=_sc_scratch(),
    name="sc_edge_agg_deg",
)

_sc_agg = pl.kernel(
    _sc_agg_nodeg_body,
    out_type=[jax.ShapeDtypeStruct((_NC, _N, _H), jnp.float32)],
    mesh=_sc_mesh,
    scratch_types=_sc_scratch(),
    name="sc_edge_agg",
)

_R = 2000
_NBLK = _N // _R


def _tc_layer_body(x_ref, agg_ref, deg_ref, wr_ref, wn_ref, b_ref, o_ref):
  agg = agg_ref[0] + agg_ref[1]
  deg = deg_ref[0] + deg_ref[1]
  mean = agg / jnp.maximum(deg, 1.0)
  h = jnp.dot(x_ref[...], wr_ref[...], preferred_element_type=jnp.float32)
  h = h + jnp.dot(mean, wn_ref[...], preferred_element_type=jnp.float32)
  o_ref[...] = jnp.maximum(h + b_ref[...], 0.0)


def _tc_layer(x, aggp, degp, W_root, W_nei, b):
  return pl.pallas_call(
      _tc_layer_body,
      grid=(_NBLK,),
      in_specs=[
          pl.BlockSpec((_R, _H), lambda i: (i, 0)),
          pl.BlockSpec((_NC, _R, _H), lambda i: (0, i, 0)),
          pl.BlockSpec((_NC, _R, 1), lambda i: (0, i, 0)),
          pl.BlockSpec((_H, _H), lambda i: (0, 0)),
          pl.BlockSpec((_H, _H), lambda i: (0, 0)),
          pl.BlockSpec((1, _H), lambda i: (0, 0)),
      ],
      out_specs=pl.BlockSpec((_R, _H), lambda i: (i, 0)),
      out_shape=jax.ShapeDtypeStruct((_N, _H), jnp.float32),
  )(x, aggp, degp, W_root, W_nei, b)


def _tc_final_body(h_ref, agg_ref, deg_ref, batch_ref,
                   wr_ref, wn_ref, b2_ref, wp1_ref, bp1_ref, wp2_ref, bp2_ref,
                   o_ref, sums, cnts):
  i = pl.program_id(0)

  @pl.when(i == 0)
  def _():
    sums[...] = jnp.zeros_like(sums)
    cnts[...] = jnp.zeros_like(cnts)

  agg = agg_ref[0] + agg_ref[1]
  deg = deg_ref[0] + deg_ref[1]
  mean = agg / jnp.maximum(deg, 1.0)
  h2 = jnp.dot(h_ref[...], wr_ref[...], preferred_element_type=jnp.float32)
  h2 = h2 + jnp.dot(mean, wn_ref[...], preferred_element_type=jnp.float32)
  h2 = jnp.maximum(h2 + b2_ref[...], 0.0)

  bt = batch_ref[0]
  gid = lax.broadcasted_iota(jnp.int32, (_G, _R), 0)
  oh = (bt == gid).astype(jnp.float32)
  sums[...] += jnp.dot(oh, h2, preferred_element_type=jnp.float32)
  cnts[...] += jnp.sum(oh, axis=1, keepdims=True)

  @pl.when(i == _NBLK - 1)
  def _():
    pooled = sums[...] / jnp.maximum(cnts[...], 1.0)
    hid = jnp.maximum(
        jnp.dot(pooled, wp1_ref[...], preferred_element_type=jnp.float32)
        + bp1_ref[...], 0.0)
    o_ref[...] = (jnp.dot(hid, wp2_ref[...], preferred_element_type=jnp.float32)
                  + bp2_ref[...])


def _tc_final(h1, aggp, degp, batch3, W_root2, W_nei2, b2, Wp1, bp1, Wp2, bp2):
  ph = Wp1.shape[1]
  return pl.pallas_call(
      _tc_final_body,
      grid=(_NBLK,),
      in_specs=[
          pl.BlockSpec((_R, _H), lambda i: (i, 0)),
          pl.BlockSpec((_NC, _R, _H), lambda i: (0, i, 0)),
          pl.BlockSpec((_NC, _R, 1), lambda i: (0, i, 0)),
          pl.BlockSpec((1, 1, _R), lambda i: (i, 0, 0)),
          pl.BlockSpec((_H, _H), lambda i: (0, 0)),
          pl.BlockSpec((_H, _H), lambda i: (0, 0)),
          pl.BlockSpec((1, _H), lambda i: (0, 0)),
          pl.BlockSpec((_H, ph), lambda i: (0, 0)),
          pl.BlockSpec((1, ph), lambda i: (0, 0)),
          pl.BlockSpec((ph, 1), lambda i: (0, 0)),
          pl.BlockSpec((1, 1), lambda i: (0, 0)),
      ],
      out_specs=pl.BlockSpec((_G, 1), lambda i: (0, 0)),
      out_shape=jax.ShapeDtypeStruct((_G, 1), jnp.float32),
      scratch_shapes=[
          pltpu.VMEM((_G, _H), jnp.float32),
          pltpu.VMEM((_G, 1), jnp.float32),
      ],
  )(h1, aggp, degp, batch3, W_root2, W_nei2, b2, Wp1, bp1, Wp2, bp2)


@jax.jit
def kernel(x, edge_index, batch, W_root1, W_nei1, b1, W_root2, W_nei2, b2,
           Wp1, bp1, Wp2, bp2):
  src = edge_index[0]
  dst = edge_index[1]
  ones_hbm = jnp.ones((_CH,), jnp.float32)
  z2d = jnp.zeros((_RPT, _H), jnp.float32)
  z1d = jnp.zeros((_DPT,), jnp.float32)

  aggp1, degp = _sc_agg_deg(x, src, dst, ones_hbm, z2d, z1d)
  degp3 = degp.reshape(_NC, _N, 1)
  h1 = _tc_layer(x, aggp1, degp3, W_root1, W_nei1, b1.reshape(1, _H))
  (aggp2,) = _sc_agg(h1, src, dst, ones_hbm, z2d, z1d)
  batch3 = batch.reshape(_NBLK, 1, _R)
  out = _tc_final(h1, aggp2, degp3, batch3, W_root2, W_nei2,
                  b2.reshape(1, _H), Wp1, bp1.reshape(1, -1),
                  Wp2, bp2.reshape(1, 1))
  return out


# speedup vs baseline: 3.2180x; 3.2180x over previous
import functools

import jax
import jax.numpy as jnp
from jax import lax
from jax.experimental import pallas as pl
from jax.experimental.pallas import tpu as pltpu
from jax.experimental.pallas import tpu_sc as plsc

_N = 10000
_E = 320000
_H = 128
_HW = 64
_G = 64

_NC = 2
_NS = 16
_NW = _NC * _NS
_EPW = _E // _NW
_CH = 80
_NCHUNK = _EPW // _CH
_RPT = 624
_RTL = _N - _NS * _RPT
_ZR = 208
_DW = 8


def _sc_agg_body(with_deg, x_hbm, src_hbm, dst_hbm, ones_hbm, zrow_hbm,
                 zdeg_hbm, agg_hbm, deg_hbm, idx_s, idx_d, rows, ones_v, zbuf,
                 zdeg, sem, shared_agg, shared_deg):
  c = lax.axis_index("c")
  s = lax.axis_index("s")
  wid = c * _NS + s

  pltpu.sync_copy(zrow_hbm, zbuf)
  for r in range(_RPT // _ZR):
    pltpu.sync_copy(zbuf, shared_agg.at[pl.ds(s * _RPT + r * _ZR, _ZR)])

  @pl.when(s == 0)
  def _():
    pltpu.sync_copy(zbuf.at[pl.ds(0, _RTL)],
                    shared_agg.at[pl.ds(_NS * _RPT, _RTL)])

  if with_deg:
    pltpu.sync_copy(zdeg_hbm, zdeg)
    pltpu.sync_copy(zdeg, shared_deg.at[pl.ds(s * _RPT, _RPT)])

    @pl.when(s == 0)
    def _():
      pltpu.sync_copy(zdeg.at[pl.ds(0, _RTL)],
                      shared_deg.at[pl.ds(_NS * _RPT, _RTL)])

    pltpu.sync_copy(ones_hbm, ones_v)
  plsc.subcore_barrier()

  base = wid * _EPW

  def chunk(i, carry):
    off = base + i * _CH
    pltpu.sync_copy(src_hbm.at[pl.ds(off, _CH)], idx_s)
    pltpu.sync_copy(dst_hbm.at[pl.ds(off, _CH)], idx_d)
    pltpu.async_copy(x_hbm.at[idx_s], rows, sem).wait()
    pltpu.sync_copy(rows, shared_agg.at[idx_d], add=True)
    if with_deg:
      pltpu.sync_copy(ones_v, shared_deg.at[idx_d], add=True)
    return carry

  lax.fori_loop(0, _NCHUNK, chunk, 0)
  plsc.subcore_barrier()

  for r in range(_RPT // _ZR):
    pltpu.sync_copy(shared_agg.at[pl.ds(s * _RPT + r * _ZR, _ZR)], zbuf)
    pltpu.sync_copy(zbuf, agg_hbm.at[c, pl.ds(s * _RPT + r * _ZR, _ZR)])

  @pl.when(s == 0)
  def _():
    pltpu.sync_copy(shared_agg.at[pl.ds(_NS * _RPT, _RTL)],
                    zbuf.at[pl.ds(0, _RTL)])
    pltpu.sync_copy(zbuf.at[pl.ds(0, _RTL)],
                    agg_hbm.at[c, pl.ds(_NS * _RPT, _RTL)])

  if with_deg:
    pltpu.sync_copy(shared_deg.at[pl.ds(s * _RPT, _RPT)], zdeg)
    pltpu.sync_copy(zdeg, deg_hbm.at[c, pl.ds(s * _RPT, _RPT)])

    @pl.when(s == 0)
    def _():
      pltpu.sync_copy(shared_deg.at[pl.ds(_NS * _RPT, _RTL)],
                      zdeg.at[pl.ds(0, _RTL)])
      pltpu.sync_copy(zdeg.at[pl.ds(0, _RTL)],
                      deg_hbm.at[c, pl.ds(_NS * _RPT, _RTL)])


def _sc_agg_deg_body(x_hbm, src_hbm, dst_hbm, ones_hbm, zrow_hbm, zdeg_hbm,
                     agg_hbm, deg_hbm, idx_s, idx_d, rows, ones_v, zbuf, zdeg,
                     sem, shared_agg, shared_deg):
  _sc_agg_body(True, x_hbm, src_hbm, dst_hbm, ones_hbm, zrow_hbm, zdeg_hbm,
               agg_hbm, deg_hbm, idx_s, idx_d, rows, ones_v, zbuf, zdeg, sem,
               shared_agg, shared_deg)


def _sc_agg_nodeg_body(x_hbm, src_hbm, dst_hbm, ones_hbm, zrow_hbm, zdeg_hbm,
                       agg_hbm, idx_s, idx_d, rows, ones_v, zbuf, zdeg,
                       sem, shared_agg, shared_deg):
  _sc_agg_body(False, x_hbm, src_hbm, dst_hbm, ones_hbm, zrow_hbm, zdeg_hbm,
               agg_hbm, None, idx_s, idx_d, rows, ones_v, zbuf, zdeg, sem,
               shared_agg, shared_deg)


def _sc_scratch():
  return [
      pltpu.VMEM((_CH,), jnp.int32),
      pltpu.VMEM((_CH,), jnp.int32),
      pltpu.VMEM((_CH, _HW), jnp.float32),
      pltpu.VMEM((_CH, _DW), jnp.float32),
      pltpu.VMEM((_ZR, _HW), jnp.float32),
      pltpu.VMEM((_RPT, _DW), jnp.float32),
      pltpu.SemaphoreType.DMA,
      pltpu.VMEM_SHARED((_N, _HW), jnp.float32),
      pltpu.VMEM_SHARED((_N, _DW), jnp.float32),
  ]


@functools.lru_cache(maxsize=None)
def _get_sc_kernels():
  mesh = plsc.VectorSubcoreMesh(core_axis_name="c", subcore_axis_name="s",
                                num_cores=_NC, num_subcores=_NS)
  agg_deg = pl.kernel(
      _sc_agg_deg_body,
      out_type=[jax.ShapeDtypeStruct((_NC, _N, _HW), jnp.float32),
                jax.ShapeDtypeStruct((_NC, _N, _DW), jnp.float32)],
      mesh=mesh,
      scratch_types=_sc_scratch(),
      compiler_params=pltpu.CompilerParams(use_tc_tiling_on_sc=False),
      name="sc_edge_agg_deg",
  )
  agg = pl.kernel(
      _sc_agg_nodeg_body,
      out_type=[jax.ShapeDtypeStruct((_NC, _N, _HW), jnp.float32)],
      mesh=mesh,
      scratch_types=_sc_scratch(),
      compiler_params=pltpu.CompilerParams(use_tc_tiling_on_sc=False),
      name="sc_edge_agg",
  )
  return agg_deg, agg

_R = 2000
_NBLK = _N // _R


def _tc_layer_body(x_ref, alo_ref, ahi_ref, deg_ref, wr_ref, wn_ref, b_ref,
                   o_ref):
  agg = jnp.concatenate([alo_ref[0] + alo_ref[1],
                         ahi_ref[0] + ahi_ref[1]], axis=1)
  deg = deg_ref[0, :, 0:1] + deg_ref[1, :, 0:1]
  mean = agg / jnp.maximum(deg, 1.0)
  h = jnp.dot(x_ref[...], wr_ref[...], preferred_element_type=jnp.float32)
  h = h + jnp.dot(mean, wn_ref[...], preferred_element_type=jnp.float32)
  o_ref[...] = jnp.maximum(h + b_ref[...], 0.0)


def _tc_layer(x, alo, ahi, degp, W_root, W_nei, b):
  return pl.pallas_call(
      _tc_layer_body,
      grid=(_NBLK,),
      in_specs=[
          pl.BlockSpec((_R, _H), lambda i: (i, 0)),
          pl.BlockSpec((_NC, _R, _HW), lambda i: (0, i, 0)),
          pl.BlockSpec((_NC, _R, _HW), lambda i: (0, i, 0)),
          pl.BlockSpec((_NC, _R, _DW), lambda i: (0, i, 0)),
          pl.BlockSpec((_H, _H), lambda i: (0, 0)),
          pl.BlockSpec((_H, _H), lambda i: (0, 0)),
          pl.BlockSpec((1, _H), lambda i: (0, 0)),
      ],
      out_specs=pl.BlockSpec((_R, _H), lambda i: (i, 0)),
      out_shape=jax.ShapeDtypeStruct((_N, _H), jnp.float32),
  )(x, alo, ahi, degp, W_root, W_nei, b)


def _tc_final_body(h_ref, alo_ref, ahi_ref, deg_ref, batch_ref,
                   wr_ref, wn_ref, b2_ref, wp1_ref, bp1_ref, wp2_ref, bp2_ref,
                   o_ref, sums, cnts):
  i = pl.program_id(0)

  @pl.when(i == 0)
  def _():
    sums[...] = jnp.zeros_like(sums)
    cnts[...] = jnp.zeros_like(cnts)

  agg = jnp.concatenate([alo_ref[0] + alo_ref[1],
                         ahi_ref[0] + ahi_ref[1]], axis=1)
  deg = deg_ref[0, :, 0:1] + deg_ref[1, :, 0:1]
  mean = agg / jnp.maximum(deg, 1.0)
  h2 = jnp.dot(h_ref[...], wr_ref[...], preferred_element_type=jnp.float32)
  h2 = h2 + jnp.dot(mean, wn_ref[...], preferred_element_type=jnp.float32)
  h2 = jnp.maximum(h2 + b2_ref[...], 0.0)

  bt = batch_ref[0]
  gid = lax.broadcasted_iota(jnp.int32, (_G, _R), 0)
  oh = (bt == gid).astype(jnp.float32)
  sums[...] += jnp.dot(oh, h2, preferred_element_type=jnp.float32)
  cnts[...] += jnp.sum(oh, axis=1, keepdims=True)

  @pl.when(i == _NBLK - 1)
  def _():
    pooled = sums[...] / jnp.maximum(cnts[...], 1.0)
    hid = jnp.maximum(
        jnp.dot(pooled, wp1_ref[...], preferred_element_type=jnp.float32)
        + bp1_ref[...], 0.0)
    o_ref[...] = (jnp.dot(hid, wp2_ref[...], preferred_element_type=jnp.float32)
                  + bp2_ref[...])


def _tc_final(h1, alo, ahi, degp, batch3, W_root2, W_nei2, b2,
              Wp1, bp1, Wp2, bp2):
  ph = Wp1.shape[1]
  return pl.pallas_call(
      _tc_final_body,
      grid=(_NBLK,),
      in_specs=[
          pl.BlockSpec((_R, _H), lambda i: (i, 0)),
          pl.BlockSpec((_NC, _R, _HW), lambda i: (0, i, 0)),
          pl.BlockSpec((_NC, _R, _HW), lambda i: (0, i, 0)),
          pl.BlockSpec((_NC, _R, _DW), lambda i: (0, i, 0)),
          pl.BlockSpec((1, 1, _R), lambda i: (i, 0, 0)),
          pl.BlockSpec((_H, _H), lambda i: (0, 0)),
          pl.BlockSpec((_H, _H), lambda i: (0, 0)),
          pl.BlockSpec((1, _H), lambda i: (0, 0)),
          pl.BlockSpec((_H, ph), lambda i: (0, 0)),
          pl.BlockSpec((1, ph), lambda i: (0, 0)),
          pl.BlockSpec((ph, 1), lambda i: (0, 0)),
          pl.BlockSpec((1, 1), lambda i: (0, 0)),
      ],
      out_specs=pl.BlockSpec((_G, 1), lambda i: (0, 0)),
      out_shape=jax.ShapeDtypeStruct((_G, 1), jnp.float32),
      scratch_shapes=[
          pltpu.VMEM((_G, _H), jnp.float32),
          pltpu.VMEM((_G, 1), jnp.float32),
      ],
  )(h1, alo, ahi, degp, batch3, W_root2, W_nei2, b2, Wp1, bp1, Wp2, bp2)


@jax.jit
def kernel(x, edge_index, batch, W_root1, W_nei1, b1, W_root2, W_nei2, b2,
           Wp1, bp1, Wp2, bp2):
  src = edge_index[0]
  dst = edge_index[1]
  ones_hbm = jnp.ones((_CH, _DW), jnp.float32)
  zrow = jnp.zeros((_ZR, _HW), jnp.float32)
  zdeg = jnp.zeros((_RPT, _DW), jnp.float32)

  sc_agg_deg, sc_agg = _get_sc_kernels()
  xlo = x[:, :_HW] + 0.0
  xhi = x[:, _HW:] + 0.0
  alo1, degp = sc_agg_deg(xlo, src, dst, ones_hbm, zrow, zdeg)
  (ahi1,) = sc_agg(xhi, src, dst, ones_hbm, zrow, zdeg)
  h1 = _tc_layer(x, alo1, ahi1, degp, W_root1, W_nei1, b1.reshape(1, _H))
  hlo = h1[:, :_HW] + 0.0
  hhi = h1[:, _HW:] + 0.0
  (alo2,) = sc_agg(hlo, src, dst, ones_hbm, zrow, zdeg)
  (ahi2,) = sc_agg(hhi, src, dst, ones_hbm, zrow, zdeg)
  batch3 = batch.reshape(_NBLK, 1, _R)
  out = _tc_final(h1, alo2, ahi2, degp, batch3, W_root2, W_nei2,
                  b2.reshape(1, _H), Wp1, bp1.reshape(1, -1),
                  Wp2, bp2.reshape(1, 1))
  return out


# speedup vs baseline: 9.6207x; 2.9896x over previous
import functools

import jax
import jax.numpy as jnp
from jax import lax
from jax.experimental import pallas as pl
from jax.experimental.pallas import tpu as pltpu
from jax.experimental.pallas import tpu_sc as plsc

_N = 10000
_E = 320000
_H = 128
_HW = 64
_G = 64

_NC = 2
_NS = 16
_EPT = _E // _NS
_CH = 100
_NCHUNK = _EPT // _CH
_NBUF = 4
_PD = 2
_RPT = 624
_RTL = _N - _NS * _RPT
_ZR = 104
_DW = 8


def _sc_agg_body(with_deg, xlo_hbm, xhi_hbm, src_hbm, dst_hbm, ones_hbm,
                 zrow_hbm, zdeg_hbm, alo_hbm, ahi_hbm, deg_hbm,
                 idx_s, idx_d, r0, r1, r2, r3, ones_v, zbuf, zdeg,
                 g0, g1, g2, g3, s0, s1, s2, s3, dsem,
                 shared_agg, shared_deg):
  c = lax.axis_index("c")
  s = lax.axis_index("s")
  rows = (r0, r1, r2, r3)
  gsem = (g0, g1, g2, g3)
  ssem = (s0, s1, s2, s3)

  pltpu.sync_copy(zrow_hbm, zbuf)
  for r in range(_RPT // _ZR):
    pltpu.sync_copy(zbuf, shared_agg.at[pl.ds(s * _RPT + r * _ZR, _ZR)])

  @pl.when(s == 0)
  def _():
    pltpu.sync_copy(zbuf.at[pl.ds(0, _RTL)],
                    shared_agg.at[pl.ds(_NS * _RPT, _RTL)])

  if with_deg:
    @pl.when(c == 0)
    def _():
      pltpu.sync_copy(zdeg_hbm, zdeg)
      pltpu.sync_copy(zdeg, shared_deg.at[pl.ds(s * _RPT, _RPT)])
      pltpu.sync_copy(ones_hbm, ones_v)

      @pl.when(s == 0)
      def _():
        pltpu.sync_copy(zdeg.at[pl.ds(0, _RTL)],
                        shared_deg.at[pl.ds(_NS * _RPT, _RTL)])

  pltpu.sync_copy(src_hbm.at[pl.ds(s * _NCHUNK, _NCHUNK)], idx_s)
  pltpu.sync_copy(dst_hbm.at[pl.ds(s * _NCHUNK, _NCHUNK)], idx_d)
  plsc.subcore_barrier()

  def start_gather(j, b):
    @pl.when(c == 0)
    def _():
      pltpu.async_copy(xlo_hbm.at[idx_s.at[j]], rows[b], gsem[b])

    @pl.when(c != 0)
    def _():
      pltpu.async_copy(xhi_hbm.at[idx_s.at[j]], rows[b], gsem[b])

  def wait_gather(j, b):
    @pl.when(c == 0)
    def _():
      pltpu.make_async_copy(xlo_hbm.at[idx_s.at[j]], rows[b], gsem[b]).wait()

    @pl.when(c != 0)
    def _():
      pltpu.make_async_copy(xhi_hbm.at[idx_s.at[j]], rows[b], gsem[b]).wait()

  def start_scatter(j, b):
    pltpu.async_copy(rows[b], shared_agg.at[idx_d.at[j]], ssem[b], add=True)
    if with_deg:
      @pl.when(c == 0)
      def _():
        pltpu.async_copy(ones_v, shared_deg.at[idx_d.at[j]], dsem, add=True)

  def wait_scatter(j, b):
    pltpu.make_async_copy(rows[b], shared_agg.at[idx_d.at[j]],
                          ssem[b]).wait()

  start_gather(0, 0)
  start_gather(1, 1)
  for b in range(_NBUF):
    i = b
    if i >= _PD:
      wait_scatter(i - _PD, (b + _PD) % _NBUF)
    wait_gather(i, b)
    start_scatter(i, b)
    start_gather(i + _PD, (b + _PD) % _NBUF)

  def group(g, carry):
    for b in range(_NBUF):
      i = g * _NBUF + b
      wait_scatter(i - _PD, (b + _PD) % _NBUF)
      wait_gather(i, b)
      start_scatter(i, b)
      start_gather(i + _PD, (b + _PD) % _NBUF)
    return carry

  lax.fori_loop(1, _NCHUNK // _NBUF - 1, group, 0)

  for b in range(_NBUF):
    i = _NCHUNK - _NBUF + b
    wait_scatter(i - _PD, (b + _PD) % _NBUF)
    wait_gather(i, b)
    start_scatter(i, b)
    if i + _PD < _NCHUNK:
      start_gather(i + _PD, (b + _PD) % _NBUF)
  wait_scatter(_NCHUNK - 2, (_NBUF - 2) % _NBUF)
  wait_scatter(_NCHUNK - 1, _NBUF - 1)

  if with_deg:
    @pl.when(c == 0)
    def _():
      def drain(i, carry):
        pltpu.make_async_copy(ones_v, shared_deg.at[idx_d.at[i]],
                              dsem).wait()
        return carry
      lax.fori_loop(0, _NCHUNK, drain, 0)

  plsc.subcore_barrier()

  out = [alo_hbm, ahi_hbm]
  for ci in range(_NC):
    @pl.when(c == ci)
    def _(ci=ci):
      for r in range(_RPT // _ZR):
        pltpu.sync_copy(shared_agg.at[pl.ds(s * _RPT + r * _ZR, _ZR)], zbuf)
        pltpu.sync_copy(zbuf, out[ci].at[pl.ds(s * _RPT + r * _ZR, _ZR)])

      @pl.when(s == 0)
      def _():
        pltpu.sync_copy(shared_agg.at[pl.ds(_NS * _RPT, _RTL)],
                        zbuf.at[pl.ds(0, _RTL)])
        pltpu.sync_copy(zbuf.at[pl.ds(0, _RTL)],
                        out[ci].at[pl.ds(_NS * _RPT, _RTL)])

  if with_deg:
    @pl.when(c == 0)
    def _():
      pltpu.sync_copy(shared_deg.at[pl.ds(s * _RPT, _RPT)], zdeg)
      pltpu.sync_copy(zdeg, deg_hbm.at[pl.ds(s * _RPT, _RPT)])

      @pl.when(s == 0)
      def _():
        pltpu.sync_copy(shared_deg.at[pl.ds(_NS * _RPT, _RTL)],
                        zdeg.at[pl.ds(0, _RTL)])
        pltpu.sync_copy(zdeg.at[pl.ds(0, _RTL)],
                        deg_hbm.at[pl.ds(_NS * _RPT, _RTL)])


def _sc_agg_deg_body(xlo_hbm, xhi_hbm, src_hbm, dst_hbm, ones_hbm, zrow_hbm,
                     zdeg_hbm, alo_hbm, ahi_hbm, deg_hbm, *rest):
  _sc_agg_body(True, xlo_hbm, xhi_hbm, src_hbm, dst_hbm, ones_hbm, zrow_hbm,
               zdeg_hbm, alo_hbm, ahi_hbm, deg_hbm, *rest)


def _sc_agg_nodeg_body(xlo_hbm, xhi_hbm, src_hbm, dst_hbm, ones_hbm, zrow_hbm,
                       zdeg_hbm, alo_hbm, ahi_hbm, *rest):
  _sc_agg_body(False, xlo_hbm, xhi_hbm, src_hbm, dst_hbm, ones_hbm, zrow_hbm,
               zdeg_hbm, alo_hbm, ahi_hbm, None, *rest)


def _sc_scratch():
  return ([
      pltpu.VMEM((_NCHUNK, _CH), jnp.int32),
      pltpu.VMEM((_NCHUNK, _CH), jnp.int32),
  ] + [pltpu.VMEM((_CH, _HW), jnp.float32) for _ in range(_NBUF)]
    + [
      pltpu.VMEM((_CH, _DW), jnp.float32),
      pltpu.VMEM((_ZR, _HW), jnp.float32),
      pltpu.VMEM((_RPT, _DW), jnp.float32),
  ] + [pltpu.SemaphoreType.DMA for _ in range(2 * _NBUF + 1)]
    + [
      pltpu.VMEM_SHARED((_N, _HW), jnp.float32),
      pltpu.VMEM_SHARED((_N, _DW), jnp.float32),
  ])


@functools.lru_cache(maxsize=None)
def _get_sc_kernels():
  mesh = plsc.VectorSubcoreMesh(core_axis_name="c", subcore_axis_name="s",
                                num_cores=_NC, num_subcores=_NS)
  agg_deg = pl.kernel(
      _sc_agg_deg_body,
      out_type=[jax.ShapeDtypeStruct((_N, _HW), jnp.float32),
                jax.ShapeDtypeStruct((_N, _HW), jnp.float32),
                jax.ShapeDtypeStruct((_N, _DW), jnp.float32)],
      mesh=mesh,
      scratch_types=_sc_scratch(),
      compiler_params=pltpu.CompilerParams(use_tc_tiling_on_sc=False),
      name="sc_edge_agg_deg",
  )
  agg = pl.kernel(
      _sc_agg_nodeg_body,
      out_type=[jax.ShapeDtypeStruct((_N, _HW), jnp.float32),
                jax.ShapeDtypeStruct((_N, _HW), jnp.float32)],
      mesh=mesh,
      scratch_types=_sc_scratch(),
      compiler_params=pltpu.CompilerParams(use_tc_tiling_on_sc=False),
      name="sc_edge_agg",
  )
  return agg_deg, agg

_R = 2000
_NBLK = _N // _R


def _tc_layer_body(x_ref, alo_ref, ahi_ref, deg_ref, wr_ref, wn_ref, b_ref,
                   o_ref):
  agg = jnp.concatenate([alo_ref[...], ahi_ref[...]], axis=1)
  deg = deg_ref[:, 0:1]
  mean = agg / jnp.maximum(deg, 1.0)
  h = jnp.dot(x_ref[...], wr_ref[...], preferred_element_type=jnp.float32)
  h = h + jnp.dot(mean, wn_ref[...], preferred_element_type=jnp.float32)
  o_ref[...] = jnp.maximum(h + b_ref[...], 0.0)


def _tc_layer(x, alo, ahi, degp, W_root, W_nei, b):
  return pl.pallas_call(
      _tc_layer_body,
      grid=(_NBLK,),
      in_specs=[
          pl.BlockSpec((_R, _H), lambda i: (i, 0)),
          pl.BlockSpec((_R, _HW), lambda i: (i, 0)),
          pl.BlockSpec((_R, _HW), lambda i: (i, 0)),
          pl.BlockSpec((_R, _DW), lambda i: (i, 0)),
          pl.BlockSpec((_H, _H), lambda i: (0, 0)),
          pl.BlockSpec((_H, _H), lambda i: (0, 0)),
          pl.BlockSpec((1, _H), lambda i: (0, 0)),
      ],
      out_specs=pl.BlockSpec((_R, _H), lambda i: (i, 0)),
      out_shape=jax.ShapeDtypeStruct((_N, _H), jnp.float32),
  )(x, alo, ahi, degp, W_root, W_nei, b)


def _tc_final_body(h_ref, alo_ref, ahi_ref, deg_ref, batch_ref,
                   wr_ref, wn_ref, b2_ref, wp1_ref, bp1_ref, wp2_ref, bp2_ref,
                   o_ref, sums, cnts):
  i = pl.program_id(0)

  @pl.when(i == 0)
  def _():
    sums[...] = jnp.zeros_like(sums)
    cnts[...] = jnp.zeros_like(cnts)

  agg = jnp.concatenate([alo_ref[...], ahi_ref[...]], axis=1)
  deg = deg_ref[:, 0:1]
  mean = agg / jnp.maximum(deg, 1.0)
  h2 = jnp.dot(h_ref[...], wr_ref[...], preferred_element_type=jnp.float32)
  h2 = h2 + jnp.dot(mean, wn_ref[...], preferred_element_type=jnp.float32)
  h2 = jnp.maximum(h2 + b2_ref[...], 0.0)

  bt = batch_ref[0]
  gid = lax.broadcasted_iota(jnp.int32, (_G, _R), 0)
  oh = (bt == gid).astype(jnp.float32)
  sums[...] += jnp.dot(oh, h2, preferred_element_type=jnp.float32)
  cnts[...] += jnp.sum(oh, axis=1, keepdims=True)

  @pl.when(i == _NBLK - 1)
  def _():
    pooled = sums[...] / jnp.maximum(cnts[...], 1.0)
    hid = jnp.maximum(
        jnp.dot(pooled, wp1_ref[...], preferred_element_type=jnp.float32)
        + bp1_ref[...], 0.0)
    o_ref[...] = (jnp.dot(hid, wp2_ref[...], preferred_element_type=jnp.float32)
                  + bp2_ref[...])


def _tc_final(h1, alo, ahi, degp, batch3, W_root2, W_nei2, b2,
              Wp1, bp1, Wp2, bp2):
  ph = Wp1.shape[1]
  return pl.pallas_call(
      _tc_final_body,
      grid=(_NBLK,),
      in_specs=[
          pl.BlockSpec((_R, _H), lambda i: (i, 0)),
          pl.BlockSpec((_R, _HW), lambda i: (i, 0)),
          pl.BlockSpec((_R, _HW), lambda i: (i, 0)),
          pl.BlockSpec((_R, _DW), lambda i: (i, 0)),
          pl.BlockSpec((1, 1, _R), lambda i: (i, 0, 0)),
          pl.BlockSpec((_H, _H), lambda i: (0, 0)),
          pl.BlockSpec((_H, _H), lambda i: (0, 0)),
          pl.BlockSpec((1, _H), lambda i: (0, 0)),
          pl.BlockSpec((_H, ph), lambda i: (0, 0)),
          pl.BlockSpec((1, ph), lambda i: (0, 0)),
          pl.BlockSpec((ph, 1), lambda i: (0, 0)),
          pl.BlockSpec((1, 1), lambda i: (0, 0)),
      ],
      out_specs=pl.BlockSpec((_G, 1), lambda i: (0, 0)),
      out_shape=jax.ShapeDtypeStruct((_G, 1), jnp.float32),
      scratch_shapes=[
          pltpu.VMEM((_G, _H), jnp.float32),
          pltpu.VMEM((_G, 1), jnp.float32),
      ],
  )(h1, alo, ahi, degp, batch3, W_root2, W_nei2, b2, Wp1, bp1, Wp2, bp2)


@jax.jit
def kernel(x, edge_index, batch, W_root1, W_nei1, b1, W_root2, W_nei2, b2,
           Wp1, bp1, Wp2, bp2):
  src = edge_index[0].reshape(_E // _CH, _CH)
  dst = edge_index[1].reshape(_E // _CH, _CH)
  ones_hbm = jnp.ones((_CH, _DW), jnp.float32)
  zrow = jnp.zeros((_ZR, _HW), jnp.float32)
  zdeg = jnp.zeros((_RPT, _DW), jnp.float32)

  sc_agg_deg, sc_agg = _get_sc_kernels()
  xlo = x[:, :_HW] + 0.0
  xhi = x[:, _HW:] + 0.0
  alo1, ahi1, degp = sc_agg_deg(xlo, xhi, src, dst, ones_hbm, zrow, zdeg)
  h1 = _tc_layer(x, alo1, ahi1, degp, W_root1, W_nei1, b1.reshape(1, _H))
  hlo = h1[:, :_HW] + 0.0
  hhi = h1[:, _HW:] + 0.0
  alo2, ahi2 = sc_agg(hlo, hhi, src, dst, ones_hbm, zrow, zdeg)
  batch3 = batch.reshape(_NBLK, 1, _R)
  out = _tc_final(h1, alo2, ahi2, degp, batch3, W_root2, W_nei2,
                  b2.reshape(1, _H), Wp1, bp1.reshape(1, -1),
                  Wp2, bp2.reshape(1, 1))
  return out


# speedup vs baseline: 9.9992x; 1.0393x over previous
import functools

import jax
import jax.numpy as jnp
from jax import lax
from jax.experimental import pallas as pl
from jax.experimental.pallas import tpu as pltpu
from jax.experimental.pallas import tpu_sc as plsc

_N = 10000
_E = 320000
_H = 128
_HW = 64
_G = 64

_NC = 2
_NS = 16
_EPT = _E // _NS
_CH = 100
_NCHUNK = _EPT // _CH
_NBUF = 4
_PD = 2
_RPT = 624
_RTL = _N - _NS * _RPT
_ZR = 104
_DW = 8


def _sc_agg_body(with_deg, xlo_hbm, xhi_hbm, src_hbm, dst_hbm, ones_hbm,
                 zrow_hbm, zdeg_hbm, alo_hbm, ahi_hbm, deg_hbm,
                 idx_s, idx_d, r0, r1, r2, r3, ones_v, zbuf, zdeg,
                 g0, g1, g2, g3, s0, s1, s2, s3, dsem,
                 shared_agg, shared_deg):
  c = lax.axis_index("c")
  s = lax.axis_index("s")
  rows = (r0, r1, r2, r3)
  gsem = (g0, g1, g2, g3)
  ssem = (s0, s1, s2, s3)

  pltpu.sync_copy(zrow_hbm, zbuf)
  for r in range(_RPT // _ZR):
    pltpu.sync_copy(zbuf, shared_agg.at[pl.ds(s * _RPT + r * _ZR, _ZR)])

  @pl.when(s == 0)
  def _():
    pltpu.sync_copy(zbuf.at[pl.ds(0, _RTL)],
                    shared_agg.at[pl.ds(_NS * _RPT, _RTL)])

  if with_deg:
    @pl.when(c == 0)
    def _():
      pltpu.sync_copy(zdeg_hbm, zdeg)
      pltpu.sync_copy(zdeg, shared_deg.at[pl.ds(s * _RPT, _RPT)])
      pltpu.sync_copy(ones_hbm, ones_v)

      @pl.when(s == 0)
      def _():
        pltpu.sync_copy(zdeg.at[pl.ds(0, _RTL)],
                        shared_deg.at[pl.ds(_NS * _RPT, _RTL)])

  pltpu.sync_copy(src_hbm.at[pl.ds(s * _NCHUNK, _NCHUNK)], idx_s)
  pltpu.sync_copy(dst_hbm.at[pl.ds(s * _NCHUNK, _NCHUNK)], idx_d)
  plsc.subcore_barrier()

  def start_gather(j, b):
    @pl.when(c == 0)
    def _():
      pltpu.async_copy(xlo_hbm.at[idx_s.at[j]], rows[b], gsem[b])

    @pl.when(c != 0)
    def _():
      pltpu.async_copy(xhi_hbm.at[idx_s.at[j]], rows[b], gsem[b])

  def wait_gather(j, b):
    @pl.when(c == 0)
    def _():
      pltpu.make_async_copy(xlo_hbm.at[idx_s.at[j]], rows[b], gsem[b]).wait()

    @pl.when(c != 0)
    def _():
      pltpu.make_async_copy(xhi_hbm.at[idx_s.at[j]], rows[b], gsem[b]).wait()

  def start_scatter(j, b):
    pltpu.async_copy(rows[b], shared_agg.at[idx_d.at[j]], ssem[b], add=True)
    if with_deg:
      @pl.when(c == 0)
      def _():
        pltpu.async_copy(ones_v, shared_deg.at[idx_d.at[j]], dsem, add=True)

  def wait_scatter(j, b):
    pltpu.make_async_copy(rows[b], shared_agg.at[idx_d.at[j]],
                          ssem[b]).wait()

  start_gather(0, 0)
  start_gather(1, 1)
  for b in range(_NBUF):
    i = b
    if i >= _PD:
      wait_scatter(i - _PD, (b + _PD) % _NBUF)
    wait_gather(i, b)
    start_scatter(i, b)
    start_gather(i + _PD, (b + _PD) % _NBUF)

  def group(g, carry):
    for b in range(_NBUF):
      i = g * _NBUF + b
      wait_scatter(i - _PD, (b + _PD) % _NBUF)
      wait_gather(i, b)
      start_scatter(i, b)
      start_gather(i + _PD, (b + _PD) % _NBUF)
    return carry

  lax.fori_loop(1, _NCHUNK // _NBUF - 1, group, 0)

  for b in range(_NBUF):
    i = _NCHUNK - _NBUF + b
    wait_scatter(i - _PD, (b + _PD) % _NBUF)
    wait_gather(i, b)
    start_scatter(i, b)
    if i + _PD < _NCHUNK:
      start_gather(i + _PD, (b + _PD) % _NBUF)
  wait_scatter(_NCHUNK - 2, (_NBUF - 2) % _NBUF)
  wait_scatter(_NCHUNK - 1, _NBUF - 1)

  if with_deg:
    @pl.when(c == 0)
    def _():
      def drain(i, carry):
        pltpu.make_async_copy(ones_v, shared_deg.at[idx_d.at[i]],
                              dsem).wait()
        return carry
      lax.fori_loop(0, _NCHUNK, drain, 0)

  plsc.subcore_barrier()

  out = [alo_hbm, ahi_hbm]
  for ci in range(_NC):
    @pl.when(c == ci)
    def _(ci=ci):
      for r in range(_RPT // _ZR):
        pltpu.sync_copy(shared_agg.at[pl.ds(s * _RPT + r * _ZR, _ZR)], zbuf)
        pltpu.sync_copy(zbuf, out[ci].at[pl.ds(s * _RPT + r * _ZR, _ZR)])

      @pl.when(s == 0)
      def _():
        pltpu.sync_copy(shared_agg.at[pl.ds(_NS * _RPT, _RTL)],
                        zbuf.at[pl.ds(0, _RTL)])
        pltpu.sync_copy(zbuf.at[pl.ds(0, _RTL)],
                        out[ci].at[pl.ds(_NS * _RPT, _RTL)])

  if with_deg:
    @pl.when(c == 0)
    def _():
      pltpu.sync_copy(shared_deg.at[pl.ds(s * _RPT, _RPT)], zdeg)
      pltpu.sync_copy(zdeg, deg_hbm.at[pl.ds(s * _RPT, _RPT)])

      @pl.when(s == 0)
      def _():
        pltpu.sync_copy(shared_deg.at[pl.ds(_NS * _RPT, _RTL)],
                        zdeg.at[pl.ds(0, _RTL)])
        pltpu.sync_copy(zdeg.at[pl.ds(0, _RTL)],
                        deg_hbm.at[pl.ds(_NS * _RPT, _RTL)])


def _sc_agg_deg_body(xlo_hbm, xhi_hbm, src_hbm, dst_hbm, ones_hbm, zrow_hbm,
                     zdeg_hbm, alo_hbm, ahi_hbm, deg_hbm, *rest):
  _sc_agg_body(True, xlo_hbm, xhi_hbm, src_hbm, dst_hbm, ones_hbm, zrow_hbm,
               zdeg_hbm, alo_hbm, ahi_hbm, deg_hbm, *rest)


def _sc_agg_nodeg_body(xlo_hbm, xhi_hbm, src_hbm, dst_hbm, ones_hbm, zrow_hbm,
                       zdeg_hbm, alo_hbm, ahi_hbm, *rest):
  _sc_agg_body(False, xlo_hbm, xhi_hbm, src_hbm, dst_hbm, ones_hbm, zrow_hbm,
               zdeg_hbm, alo_hbm, ahi_hbm, None, *rest)


def _sc_scratch():
  return ([
      pltpu.VMEM((_NCHUNK, _CH), jnp.int32),
      pltpu.VMEM((_NCHUNK, _CH), jnp.int32),
  ] + [pltpu.VMEM((_CH, _HW), jnp.float32) for _ in range(_NBUF)]
    + [
      pltpu.VMEM((_CH, _DW), jnp.float32),
      pltpu.VMEM((_ZR, _HW), jnp.float32),
      pltpu.VMEM((_RPT, _DW), jnp.float32),
  ] + [pltpu.SemaphoreType.DMA for _ in range(2 * _NBUF + 1)]
    + [
      pltpu.VMEM_SHARED((_N, _HW), jnp.float32),
      pltpu.VMEM_SHARED((_N, _DW), jnp.float32),
  ])


@functools.lru_cache(maxsize=None)
def _get_sc_kernels():
  mesh = plsc.VectorSubcoreMesh(core_axis_name="c", subcore_axis_name="s",
                                num_cores=_NC, num_subcores=_NS)
  agg_deg = pl.kernel(
      _sc_agg_deg_body,
      out_type=[jax.ShapeDtypeStruct((_N, _HW), jnp.float32),
                jax.ShapeDtypeStruct((_N, _HW), jnp.float32),
                jax.ShapeDtypeStruct((_N, _DW), jnp.float32)],
      mesh=mesh,
      scratch_types=_sc_scratch(),
      compiler_params=pltpu.CompilerParams(use_tc_tiling_on_sc=False),
      name="sc_edge_agg_deg",
  )
  agg = pl.kernel(
      _sc_agg_nodeg_body,
      out_type=[jax.ShapeDtypeStruct((_N, _HW), jnp.float32),
                jax.ShapeDtypeStruct((_N, _HW), jnp.float32)],
      mesh=mesh,
      scratch_types=_sc_scratch(),
      compiler_params=pltpu.CompilerParams(use_tc_tiling_on_sc=False),
      name="sc_edge_agg",
  )
  return agg_deg, agg

_R = 2000
_NBLK = _N // _R


def _tc_layer_body(xlo_ref, xhi_ref, alo_ref, ahi_ref, deg_ref, wr_ref,
                   wn_ref, b_ref, olo_ref, ohi_ref):
  x = jnp.concatenate([xlo_ref[...], xhi_ref[...]], axis=1)
  agg = jnp.concatenate([alo_ref[...], ahi_ref[...]], axis=1)
  deg = deg_ref[:, 0:1]
  mean = agg / jnp.maximum(deg, 1.0)
  h = jnp.dot(x, wr_ref[...], preferred_element_type=jnp.float32)
  h = h + jnp.dot(mean, wn_ref[...], preferred_element_type=jnp.float32)
  h = jnp.maximum(h + b_ref[...], 0.0)
  olo_ref[...] = h[:, :_HW]
  ohi_ref[...] = h[:, _HW:]


def _tc_layer(xlo, xhi, alo, ahi, degp, W_root, W_nei, b):
  return pl.pallas_call(
      _tc_layer_body,
      grid=(_NBLK,),
      in_specs=[
          pl.BlockSpec((_R, _HW), lambda i: (i, 0)),
          pl.BlockSpec((_R, _HW), lambda i: (i, 0)),
          pl.BlockSpec((_R, _HW), lambda i: (i, 0)),
          pl.BlockSpec((_R, _HW), lambda i: (i, 0)),
          pl.BlockSpec((_R, _DW), lambda i: (i, 0)),
          pl.BlockSpec((_H, _H), lambda i: (0, 0)),
          pl.BlockSpec((_H, _H), lambda i: (0, 0)),
          pl.BlockSpec((1, _H), lambda i: (0, 0)),
      ],
      out_specs=[pl.BlockSpec((_R, _HW), lambda i: (i, 0)),
                 pl.BlockSpec((_R, _HW), lambda i: (i, 0))],
      out_shape=[jax.ShapeDtypeStruct((_N, _HW), jnp.float32),
                 jax.ShapeDtypeStruct((_N, _HW), jnp.float32)],
  )(xlo, xhi, alo, ahi, degp, W_root, W_nei, b)


def _tc_final_body(hlo_ref, hhi_ref, alo_ref, ahi_ref, deg_ref, batch_ref,
                   wr_ref, wn_ref, b2_ref, wp1_ref, bp1_ref, wp2_ref, bp2_ref,
                   o_ref, sums, cnts):
  i = pl.program_id(0)

  @pl.when(i == 0)
  def _():
    sums[...] = jnp.zeros_like(sums)
    cnts[...] = jnp.zeros_like(cnts)

  h1 = jnp.concatenate([hlo_ref[...], hhi_ref[...]], axis=1)
  agg = jnp.concatenate([alo_ref[...], ahi_ref[...]], axis=1)
  deg = deg_ref[:, 0:1]
  mean = agg / jnp.maximum(deg, 1.0)
  h2 = jnp.dot(h1, wr_ref[...], preferred_element_type=jnp.float32)
  h2 = h2 + jnp.dot(mean, wn_ref[...], preferred_element_type=jnp.float32)
  h2 = jnp.maximum(h2 + b2_ref[...], 0.0)

  bt = batch_ref[0]
  gid = lax.broadcasted_iota(jnp.int32, (_G, _R), 0)
  oh = (bt == gid).astype(jnp.float32)
  sums[...] += jnp.dot(oh, h2, preferred_element_type=jnp.float32)
  cnts[...] += jnp.sum(oh, axis=1, keepdims=True)

  @pl.when(i == _NBLK - 1)
  def _():
    pooled = sums[...] / jnp.maximum(cnts[...], 1.0)
    hid = jnp.maximum(
        jnp.dot(pooled, wp1_ref[...], preferred_element_type=jnp.float32)
        + bp1_ref[...], 0.0)
    o_ref[...] = (jnp.dot(hid, wp2_ref[...], preferred_element_type=jnp.float32)
                  + bp2_ref[...])


def _tc_final(hlo, hhi, alo, ahi, degp, batch3, W_root2, W_nei2, b2,
              Wp1, bp1, Wp2, bp2):
  ph = Wp1.shape[1]
  return pl.pallas_call(
      _tc_final_body,
      grid=(_NBLK,),
      in_specs=[
          pl.BlockSpec((_R, _HW), lambda i: (i, 0)),
          pl.BlockSpec((_R, _HW), lambda i: (i, 0)),
          pl.BlockSpec((_R, _HW), lambda i: (i, 0)),
          pl.BlockSpec((_R, _HW), lambda i: (i, 0)),
          pl.BlockSpec((_R, _DW), lambda i: (i, 0)),
          pl.BlockSpec((1, 1, _R), lambda i: (i, 0, 0)),
          pl.BlockSpec((_H, _H), lambda i: (0, 0)),
          pl.BlockSpec((_H, _H), lambda i: (0, 0)),
          pl.BlockSpec((1, _H), lambda i: (0, 0)),
          pl.BlockSpec((_H, ph), lambda i: (0, 0)),
          pl.BlockSpec((1, ph), lambda i: (0, 0)),
          pl.BlockSpec((ph, 1), lambda i: (0, 0)),
          pl.BlockSpec((1, 1), lambda i: (0, 0)),
      ],
      out_specs=pl.BlockSpec((_G, 1), lambda i: (0, 0)),
      out_shape=jax.ShapeDtypeStruct((_G, 1), jnp.float32),
      scratch_shapes=[
          pltpu.VMEM((_G, _H), jnp.float32),
          pltpu.VMEM((_G, 1), jnp.float32),
      ],
  )(hlo, hhi, alo, ahi, degp, batch3, W_root2, W_nei2, b2, Wp1, bp1, Wp2, bp2)


@jax.jit
def kernel(x, edge_index, batch, W_root1, W_nei1, b1, W_root2, W_nei2, b2,
           Wp1, bp1, Wp2, bp2):
  src = edge_index[0].reshape(_E // _CH, _CH)
  dst = edge_index[1].reshape(_E // _CH, _CH)
  ones_hbm = jnp.ones((_CH, _DW), jnp.float32)
  zrow = jnp.zeros((_ZR, _HW), jnp.float32)
  zdeg = jnp.zeros((_RPT, _DW), jnp.float32)

  sc_agg_deg, sc_agg = _get_sc_kernels()
  xlo = x[:, :_HW] + 0.0
  xhi = x[:, _HW:] + 0.0
  alo1, ahi1, degp = sc_agg_deg(xlo, xhi, src, dst, ones_hbm, zrow, zdeg)
  hlo, hhi = _tc_layer(xlo, xhi, alo1, ahi1, degp, W_root1, W_nei1,
                       b1.reshape(1, _H))
  alo2, ahi2 = sc_agg(hlo, hhi, src, dst, ones_hbm, zrow, zdeg)
  batch3 = batch.reshape(_NBLK, 1, _R)
  out = _tc_final(hlo, hhi, alo2, ahi2, degp, batch3, W_root2, W_nei2,
                  b2.reshape(1, _H), Wp1, bp1.reshape(1, -1),
                  Wp2, bp2.reshape(1, 1))
  return out
